# baseline (speedup 1.0000x reference)
import functools

import jax
import jax.numpy as jnp
from jax import lax
from jax.experimental import pallas as pl
from jax.experimental.pallas import tpu as pltpu
from jax.experimental.pallas import tpu_sc as plsc

NC = 2
NS = 16
NW = NC * NS
K = 128


def _sc_mesh():
    return plsc.VectorSubcoreMesh(
        core_axis_name="c", subcore_axis_name="s", num_cores=NC, num_subcores=NS
    )


def _pad_rows(n):
    return ((n // NS + 7) // 8) * 8


def _make_deg_kernel(n, ch):
    rps = _pad_rows(n)
    npad = rps * NS
    cpw = ch // NW

    @functools.partial(
        pl.kernel,
        out_type=jax.ShapeDtypeStruct((NC, npad, 128), jnp.float32),
        mesh=_sc_mesh(),
        scratch_types=[
            pltpu.VMEM_SHARED((npad, 128), jnp.float32),
            pltpu.VMEM((cpw, K), jnp.int32),
            pltpu.VMEM((K, 128), jnp.float32),
        ],
    )
    def deg_kernel(dst2_hbm, zeros_hbm, ones_hbm, out_hbm, acc, dst_all, ones_v):
        cid = lax.axis_index("c")
        sid = lax.axis_index("s")
        gwid = cid * NS + sid
        pltpu.sync_copy(dst2_hbm.at[pl.ds(gwid * cpw, cpw)], dst_all)
        pltpu.sync_copy(zeros_hbm, acc.at[pl.ds(sid * rps, rps)])
        pltpu.sync_copy(ones_hbm, ones_v)
        plsc.subcore_barrier()

        def body(j, carry):
            pltpu.sync_copy(ones_v, acc.at[dst_all.at[j]], add=True)
            return carry

        lax.fori_loop(0, cpw, body, 0)
        plsc.subcore_barrier()
        pltpu.sync_copy(
            acc.at[pl.ds(sid * rps, rps)], out_hbm.at[cid, pl.ds(sid * rps, rps)]
        )

    return deg_kernel


def _make_agg_kernel(n, ch, d):
    rps = _pad_rows(n)
    npad = rps * NS
    cpw = ch // NW

    @functools.partial(
        pl.kernel,
        out_type=jax.ShapeDtypeStruct((NC, npad, d), jnp.float32),
        mesh=_sc_mesh(),
        scratch_types=[
            pltpu.VMEM_SHARED((npad, d), jnp.float32),
            pltpu.VMEM((K,), jnp.int32),
            pltpu.VMEM((K,), jnp.int32),
            pltpu.VMEM((K, d), jnp.float32),
            pltpu.SemaphoreType.DMA,
        ],
    )
    def agg_kernel(ys_hbm, src_hbm, dst_hbm, zeros_hbm, out_hbm,
                   acc, srcb, dstb, rows, sem):
        cid = lax.axis_index("c")
        sid = lax.axis_index("s")
        gwid = cid * NS + sid
        pltpu.sync_copy(zeros_hbm, acc.at[pl.ds(sid * rps, rps)])
        plsc.subcore_barrier()
        nw = (ch - gwid + NW - 1) // NW

        def body(i, carry):
            ci = (gwid + i * NW) * K
            pltpu.sync_copy(src_hbm.at[pl.ds(ci, K)], srcb)
            pltpu.sync_copy(dst_hbm.at[pl.ds(ci, K)], dstb)
            pltpu.async_copy(ys_hbm.at[srcb], rows, sem).wait()
            pltpu.sync_copy(rows, acc.at[dstb], add=True)
            return carry

        lax.fori_loop(0, nw, body, 0)
        plsc.subcore_barrier()
        pltpu.sync_copy(
            acc.at[pl.ds(sid * rps, rps)], out_hbm.at[cid, pl.ds(sid * rps, rps)]
        )

    return agg_kernel


BN = 400


def _tc0_body(x_ref, w_ref, xw_ref):
    xw_ref[...] = jnp.dot(x_ref[...], w_ref[...],
                          preferred_element_type=jnp.float32)


def _tc1_body(degp_ref, xw_ref, b_ref, ys_ref, p_ref, dis_ref):
    deg = degp_ref[0, :, 0:1] + degp_ref[1, :, 0:1] + 1.0
    dis = lax.rsqrt(jnp.maximum(deg, 1.0))
    xw = xw_ref[...]
    ys_ref[...] = dis * xw
    p_ref[...] = (dis * dis) * xw + b_ref[...]
    dis_ref[...] = dis


def _tc_mid_body(acc_ref, p_ref, dis_ref, w_ref, b_ref, ys_ref, p2_ref):
    d = dis_ref[...]
    h = jnp.maximum(d * (acc_ref[0] + acc_ref[1]) + p_ref[...], 0.0)
    xw = jnp.dot(h, w_ref[...], preferred_element_type=jnp.float32)
    ys_ref[...] = d * xw
    p2_ref[...] = (d * d) * xw + b_ref[...]


def _tc_final_body(acc_ref, p_ref, dis_ref, w_ref, b_ref, z_ref):
    d = dis_ref[...]
    h = jnp.maximum(d * (acc_ref[0] + acc_ref[1]) + p_ref[...], 0.0)
    z_ref[...] = (
        jnp.dot(h, w_ref[...], preferred_element_type=jnp.float32) + b_ref[...]
    )


def _full_spec(shape):
    return pl.BlockSpec(shape, lambda i: tuple(0 for _ in shape))


def kernel(x, edge_index, W1, b1, W2, b2, Wl, bl):
    n, d_in = x.shape
    e = edge_index.shape[1]
    h1 = W1.shape[1]
    h2 = W2.shape[1]
    d_out = Wl.shape[1]
    assert n % BN == 0

    assert e % K == 0
    rps = _pad_rows(n)
    npad = rps * NS
    src = edge_index[0]
    dst = edge_index[1]
    chunk_quant = NW * 8 * K
    e_pad = ((e + chunk_quant - 1) // chunk_quant) * chunk_quant
    ch = e_pad // K
    dummy_dst = n + jnp.arange(e_pad - e, dtype=jnp.int32) % (npad - n)
    dst_pad = jnp.concatenate([dst, dummy_dst]).reshape(ch, K)
    zerosd = jnp.zeros((rps, h1), jnp.float32)
    ones128 = jnp.ones((K, 128), jnp.float32)

    grid = n // BN
    tc0 = pl.pallas_call(
        _tc0_body,
        grid=(grid,),
        in_specs=[
            pl.BlockSpec((BN, d_in), lambda i: (i, 0)),
            _full_spec((d_in, h1)),
        ],
        out_specs=pl.BlockSpec((BN, h1), lambda i: (i, 0)),
        out_shape=jax.ShapeDtypeStruct((n, h1), jnp.float32),
    )
    xw1 = tc0(x, W1)
    degp = _make_deg_kernel(n, ch)(dst_pad, zerosd, ones128)

    tc1 = pl.pallas_call(
        _tc1_body,
        grid=(grid,),
        in_specs=[
            pl.BlockSpec((NC, BN, 128), lambda i: (0, i, 0)),
            pl.BlockSpec((BN, h1), lambda i: (i, 0)),
            _full_spec((1, h1)),
        ],
        out_specs=[
            pl.BlockSpec((BN, h1), lambda i: (i, 0)),
            pl.BlockSpec((BN, h1), lambda i: (i, 0)),
            pl.BlockSpec((BN, 1), lambda i: (i, 0)),
        ],
        out_shape=[
            jax.ShapeDtypeStruct((n, h1), jnp.float32),
            jax.ShapeDtypeStruct((n, h1), jnp.float32),
            jax.ShapeDtypeStruct((n, 1), jnp.float32),
        ],
    )
    ys1, p1, dis = tc1(degp, xw1, b1.reshape(1, h1))

    acc1 = _make_agg_kernel(n, e // K, h1)(ys1, src, dst, zerosd)

    tc2 = pl.pallas_call(
        _tc_mid_body,
        grid=(grid,),
        in_specs=[
            pl.BlockSpec((NC, BN, h1), lambda i: (0, i, 0)),
            pl.BlockSpec((BN, h1), lambda i: (i, 0)),
            pl.BlockSpec((BN, 1), lambda i: (i, 0)),
            _full_spec((h1, h2)),
            _full_spec((1, h2)),
        ],
        out_specs=[
            pl.BlockSpec((BN, h2), lambda i: (i, 0)),
            pl.BlockSpec((BN, h2), lambda i: (i, 0)),
        ],
        out_shape=[
            jax.ShapeDtypeStruct((n, h2), jnp.float32),
            jax.ShapeDtypeStruct((n, h2), jnp.float32),
        ],
    )
    ys2, p2 = tc2(acc1, p1, dis, W2, b2.reshape(1, h2))

    acc2 = _make_agg_kernel(n, e // K, h2)(ys2, src, dst, zerosd)

    tc3 = pl.pallas_call(
        _tc_final_body,
        grid=(grid,),
        in_specs=[
            pl.BlockSpec((NC, BN, h2), lambda i: (0, i, 0)),
            pl.BlockSpec((BN, h2), lambda i: (i, 0)),
            pl.BlockSpec((BN, 1), lambda i: (i, 0)),
            _full_spec((h2, d_out)),
            _full_spec((1, d_out)),
        ],
        out_specs=[pl.BlockSpec((BN, d_out), lambda i: (i, 0))],
        out_shape=[jax.ShapeDtypeStruct((n, d_out), jnp.float32)],
    )
    (z,) = tc3(acc2, p2, dis, Wl, bl.reshape(1, d_out))
    return z


# speedup vs baseline: 1.0036x; 1.0036x over previous
import functools

import jax
import jax.numpy as jnp
from jax import lax
from jax.experimental import pallas as pl
from jax.experimental.pallas import tpu as pltpu
from jax.experimental.pallas import tpu_sc as plsc

NC = 2
NS = 16
NW = NC * NS
K = 128


def _sc_mesh():
    return plsc.VectorSubcoreMesh(
        core_axis_name="c", subcore_axis_name="s", num_cores=NC, num_subcores=NS
    )


def _pad_rows(n):
    return ((n // NS + 7) // 8) * 8


def _make_deg_kernel(n, ch):
    rps = _pad_rows(n)
    npad = rps * NS
    cpw = ch // NW

    @functools.partial(
        pl.kernel,
        out_type=jax.ShapeDtypeStruct((NC, npad, 128), jnp.float32),
        mesh=_sc_mesh(),
        scratch_types=[
            pltpu.VMEM_SHARED((npad, 128), jnp.float32),
            pltpu.VMEM((cpw, K), jnp.int32),
            pltpu.VMEM((K, 128), jnp.float32),
        ],
    )
    def deg_kernel(dst2_hbm, zeros_hbm, ones_hbm, out_hbm, acc, dst_all, ones_v):
        cid = lax.axis_index("c")
        sid = lax.axis_index("s")
        gwid = cid * NS + sid
        pltpu.sync_copy(dst2_hbm.at[pl.ds(gwid * cpw, cpw)], dst_all)
        pltpu.sync_copy(zeros_hbm, acc.at[pl.ds(sid * rps, rps)])
        pltpu.sync_copy(ones_hbm, ones_v)
        plsc.subcore_barrier()

        def body(j, carry):
            pltpu.sync_copy(ones_v, acc.at[dst_all.at[j]], add=True)
            return carry

        lax.fori_loop(0, cpw, body, 0)
        plsc.subcore_barrier()
        pltpu.sync_copy(
            acc.at[pl.ds(sid * rps, rps)], out_hbm.at[cid, pl.ds(sid * rps, rps)]
        )

    return deg_kernel


def _make_agg_kernel(n, ch, d):
    rps = _pad_rows(n)
    npad = rps * NS
    cpw = ch // NW

    @functools.partial(
        pl.kernel,
        out_type=jax.ShapeDtypeStruct((NC, npad, d), jnp.float32),
        mesh=_sc_mesh(),
        scratch_types=[
            pltpu.VMEM_SHARED((npad, d), jnp.float32),
            pltpu.VMEM((K,), jnp.int32),
            pltpu.VMEM((K,), jnp.int32),
            pltpu.VMEM((K, d), jnp.float32),
            pltpu.SemaphoreType.DMA,
        ],
    )
    def agg_kernel(ys_hbm, src_hbm, dst_hbm, zeros_hbm, out_hbm,
                   acc, srcb, dstb, rows, sem):
        cid = lax.axis_index("c")
        sid = lax.axis_index("s")
        gwid = cid * NS + sid
        pltpu.sync_copy(zeros_hbm, acc.at[pl.ds(sid * rps, rps)])
        plsc.subcore_barrier()
        nw = (ch - gwid + NW - 1) // NW

        def body(i, carry):
            ci = (gwid + i * NW) * K
            pltpu.sync_copy(src_hbm.at[pl.ds(ci, K)], srcb)
            pltpu.sync_copy(dst_hbm.at[pl.ds(ci, K)], dstb)
            pltpu.async_copy(ys_hbm.at[srcb], rows, sem).wait()
            pltpu.sync_copy(rows, acc.at[dstb], add=True)
            return carry

        lax.fori_loop(0, nw, body, 0)
        plsc.subcore_barrier()
        pltpu.sync_copy(
            acc.at[pl.ds(sid * rps, rps)], out_hbm.at[cid, pl.ds(sid * rps, rps)]
        )

    return agg_kernel


BN = 400


def _tc1_body(degp_ref, x_ref, w_ref, b_ref, ys_ref, p_ref, dis_ref):
    deg = degp_ref[0, :, 0:1] + degp_ref[1, :, 0:1] + 1.0
    dis = lax.rsqrt(jnp.maximum(deg, 1.0))
    xw = jnp.dot(x_ref[...], w_ref[...], preferred_element_type=jnp.float32)
    ys_ref[...] = dis * xw
    p_ref[...] = (dis * dis) * xw + b_ref[...]
    dis_ref[...] = dis


def _tc_mid_body(acc_ref, p_ref, dis_ref, w_ref, b_ref, ys_ref, p2_ref):
    d = dis_ref[...]
    h = jnp.maximum(d * (acc_ref[0] + acc_ref[1]) + p_ref[...], 0.0)
    xw = jnp.dot(h, w_ref[...], preferred_element_type=jnp.float32)
    ys_ref[...] = d * xw
    p2_ref[...] = (d * d) * xw + b_ref[...]


def _tc_final_body(acc_ref, p_ref, dis_ref, w_ref, b_ref, z_ref):
    d = dis_ref[...]
    h = jnp.maximum(d * (acc_ref[0] + acc_ref[1]) + p_ref[...], 0.0)
    z_ref[...] = (
        jnp.dot(h, w_ref[...], preferred_element_type=jnp.float32) + b_ref[...]
    )


def _full_spec(shape):
    return pl.BlockSpec(shape, lambda i: tuple(0 for _ in shape))


def kernel(x, edge_index, W1, b1, W2, b2, Wl, bl):
    n, d_in = x.shape
    e = edge_index.shape[1]
    h1 = W1.shape[1]
    h2 = W2.shape[1]
    d_out = Wl.shape[1]
    assert n % BN == 0

    assert e % K == 0
    rps = _pad_rows(n)
    npad = rps * NS
    src = edge_index[0]
    dst = edge_index[1]
    chunk_quant = NW * 8 * K
    e_pad = ((e + chunk_quant - 1) // chunk_quant) * chunk_quant
    ch = e_pad // K
    dummy_dst = n + jnp.arange(e_pad - e, dtype=jnp.int32) % (npad - n)
    dst_pad = jnp.concatenate([dst, dummy_dst]).reshape(ch, K)
    zerosd = jnp.zeros((rps, h1), jnp.float32)
    ones128 = jnp.ones((K, 128), jnp.float32)

    degp = _make_deg_kernel(n, ch)(dst_pad, zerosd, ones128)

    grid = n // BN
    tc1 = pl.pallas_call(
        _tc1_body,
        grid=(grid,),
        in_specs=[
            pl.BlockSpec((NC, BN, 128), lambda i: (0, i, 0)),
            pl.BlockSpec((BN, d_in), lambda i: (i, 0)),
            _full_spec((d_in, h1)),
            _full_spec((1, h1)),
        ],
        out_specs=[
            pl.BlockSpec((BN, h1), lambda i: (i, 0)),
            pl.BlockSpec((BN, h1), lambda i: (i, 0)),
            pl.BlockSpec((BN, 1), lambda i: (i, 0)),
        ],
        out_shape=[
            jax.ShapeDtypeStruct((n, h1), jnp.float32),
            jax.ShapeDtypeStruct((n, h1), jnp.float32),
            jax.ShapeDtypeStruct((n, 1), jnp.float32),
        ],
    )
    ys1, p1, dis = tc1(degp, x, W1, b1.reshape(1, h1))

    acc1 = _make_agg_kernel(n, e // K, h1)(ys1, src, dst, zerosd)

    tc2 = pl.pallas_call(
        _tc_mid_body,
        grid=(grid,),
        in_specs=[
            pl.BlockSpec((NC, BN, h1), lambda i: (0, i, 0)),
            pl.BlockSpec((BN, h1), lambda i: (i, 0)),
            pl.BlockSpec((BN, 1), lambda i: (i, 0)),
            _full_spec((h1, h2)),
            _full_spec((1, h2)),
        ],
        out_specs=[
            pl.BlockSpec((BN, h2), lambda i: (i, 0)),
            pl.BlockSpec((BN, h2), lambda i: (i, 0)),
        ],
        out_shape=[
            jax.ShapeDtypeStruct((n, h2), jnp.float32),
            jax.ShapeDtypeStruct((n, h2), jnp.float32),
        ],
    )
    ys2, p2 = tc2(acc1, p1, dis, W2, b2.reshape(1, h2))

    acc2 = _make_agg_kernel(n, e // K, h2)(ys2, src, dst, zerosd)

    tc3 = pl.pallas_call(
        _tc_final_body,
        grid=(grid,),
        in_specs=[
            pl.BlockSpec((NC, BN, h2), lambda i: (0, i, 0)),
            pl.BlockSpec((BN, h2), lambda i: (i, 0)),
            pl.BlockSpec((BN, 1), lambda i: (i, 0)),
            _full_spec((h2, d_out)),
            _full_spec((1, d_out)),
        ],
        out_specs=[pl.BlockSpec((BN, d_out), lambda i: (i, 0))],
        out_shape=[jax.ShapeDtypeStruct((n, d_out), jnp.float32)],
    )
    (z,) = tc3(acc2, p2, dis, Wl, bl.reshape(1, d_out))
    return z
